# Initial kernel scaffold; baseline (speedup 1.0000x reference)
import jax
import jax.numpy as jnp
from jax import lax
from jax.experimental import pallas as pl
from jax.experimental.pallas import tpu as pltpu
from jax.experimental.pallas import tpu_sc as plsc

N_LEVELS = 10
F = 16
N = 100000

C = 400
NCHUNKS = N // C
NW = 32
LANES = 16


def _heal_body(pix_hbm, w_hbm, table_hbm, out_hbm,
               pix_v, w_v, idx_v, rows_v, out_v, sem):
    wid = lax.axis_index("s") * 2 + lax.axis_index("c")
    niter = (NCHUNKS - wid + NW - 1) // NW

    iota = lax.iota(jnp.int32, LANES)
    col_base = iota * N_LEVELS

    def chunk_body(it, _):
        c = wid + it * NW
        base = c * C

        def level_body(l, _):
            start = ((jnp.int32(1) << (2 * l)) - 1) * 4
            pltpu.sync_copy(pix_hbm.at[l, :, pl.ds(base, C)], pix_v)
            pltpu.sync_copy(w_hbm.at[l, :, pl.ds(base, C)], w_v)

            def add_body(i, _):
                for j in range(4):
                    v = pix_v[j, pl.ds(i * LANES, LANES)]
                    idx_v[pl.ds(j * C + i * LANES, LANES)] = v + start
                return 0
            lax.fori_loop(0, C // LANES, add_body, 0)

            pltpu.async_copy(table_hbm.at[idx_v], rows_v, sem).wait()

            col_ids = col_base + l

            def group_body(g, _):
                b16 = g * LANES
                wv = [w_v[j, pl.ds(b16, LANES)] for j in range(4)]
                for p in range(LANES):
                    n = b16 + p
                    acc = wv[0][p] * rows_v[n]
                    acc = acc + wv[1][p] * rows_v[C + n]
                    acc = acc + wv[2][p] * rows_v[2 * C + n]
                    acc = acc + wv[3][p] * rows_v[3 * C + n]
                    row_ids = jnp.full((LANES,), n, dtype=jnp.int32)
                    plsc.store_scatter(out_v, [row_ids, col_ids], acc)
                return 0
            lax.fori_loop(0, C // LANES, group_body, 0)
            return 0

        lax.fori_loop(0, N_LEVELS, level_body, 0)
        pltpu.sync_copy(out_v, out_hbm.at[pl.ds(base, C)])
        return 0

    lax.fori_loop(0, niter, chunk_body, 0)


def _heal_sc(params, neigh_pix, neigh_weight):
    mesh = plsc.VectorSubcoreMesh(core_axis_name="c", subcore_axis_name="s")
    kfn = pl.kernel(
        _heal_body,
        mesh=mesh,
        out_type=jax.ShapeDtypeStruct((N, F * N_LEVELS), jnp.float32),
        scratch_types=[
            pltpu.VMEM((4, C), jnp.int32),
            pltpu.VMEM((4, C), jnp.float32),
            pltpu.VMEM((4 * C,), jnp.int32),
            pltpu.VMEM((4 * C, F), jnp.float32),
            pltpu.VMEM((C, F * N_LEVELS), jnp.float32),
            pltpu.SemaphoreType.DMA,
        ],
    )
    return kfn(neigh_pix, neigh_weight, params)


def kernel(x, params, neigh_pix, neigh_weight):
    del x
    return _heal_sc(params, neigh_pix, neigh_weight)


# speedup vs baseline: 2.3174x; 2.3174x over previous
import jax
import jax.numpy as jnp
from jax import lax
from jax.experimental import pallas as pl
from jax.experimental.pallas import tpu as pltpu
from jax.experimental.pallas import tpu_sc as plsc

N_LEVELS = 10
F = 16
N = 100000

C = 400
NCHUNKS = N // C
NW = 32
LANES = 16


def _heal_body(pix_hbm, w_hbm, table_hbm, out_hbm,
               pix_v, w_v, idx_v, rows_v, out_v, sem):
    wid = lax.axis_index("s") * 2 + lax.axis_index("c")
    niter = (NCHUNKS - wid + NW - 1) // NW

    iota = lax.iota(jnp.int32, LANES)
    col_base = iota * N_LEVELS

    def chunk_body(it, _):
        c = wid + it * NW
        base = c * C

        def level_body(l, _):
            start = ((jnp.int32(1) << (2 * l)) - 1) * 4
            for j in range(4):
                off = (l * 4 + j) * N + base
                pltpu.sync_copy(pix_hbm.at[pl.ds(off, C)], pix_v.at[j])
                pltpu.sync_copy(w_hbm.at[pl.ds(off, C)], w_v.at[j])

            def add_body(i, _):
                for j in range(4):
                    v = pix_v[j, pl.ds(i * LANES, LANES)]
                    idx_v[pl.ds(j * C + i * LANES, LANES)] = v + start
                return 0
            lax.fori_loop(0, C // LANES, add_body, 0)

            pltpu.async_copy(table_hbm.at[idx_v], rows_v, sem).wait()

            col_ids = col_base + l

            def group_body(g, _):
                b16 = g * LANES
                wv = [w_v[j, pl.ds(b16, LANES)] for j in range(4)]
                for p in range(LANES):
                    n = b16 + p
                    acc = wv[0][p] * rows_v[n]
                    acc = acc + wv[1][p] * rows_v[C + n]
                    acc = acc + wv[2][p] * rows_v[2 * C + n]
                    acc = acc + wv[3][p] * rows_v[3 * C + n]
                    flat_ids = col_ids + n * (F * N_LEVELS)
                    plsc.store_scatter(out_v, [flat_ids], acc)
                return 0
            lax.fori_loop(0, C // LANES, group_body, 0)
            return 0

        lax.fori_loop(0, N_LEVELS, level_body, 0)
        pltpu.sync_copy(out_v, out_hbm.at[pl.ds(base * (F * N_LEVELS), C * (F * N_LEVELS))])
        return 0

    lax.fori_loop(0, niter, chunk_body, 0)


def _heal_sc(params, neigh_pix, neigh_weight):
    mesh = plsc.VectorSubcoreMesh(core_axis_name="c", subcore_axis_name="s")
    kfn = pl.kernel(
        _heal_body,
        mesh=mesh,
        out_type=jax.ShapeDtypeStruct((N * F * N_LEVELS,), jnp.float32),
        scratch_types=[
            pltpu.VMEM((4, C), jnp.int32),
            pltpu.VMEM((4, C), jnp.float32),
            pltpu.VMEM((4 * C,), jnp.int32),
            pltpu.VMEM((4 * C, F), jnp.float32),
            pltpu.VMEM((C * F * N_LEVELS,), jnp.float32),
            pltpu.SemaphoreType.DMA,
        ],
        compiler_params=pltpu.CompilerParams(
            use_tc_tiling_on_sc=False, needs_layout_passes=False),
    )
    out = kfn(neigh_pix.reshape(-1), neigh_weight.reshape(-1), params)
    return out.reshape(N, F * N_LEVELS)


def kernel(x, params, neigh_pix, neigh_weight):
    del x
    return _heal_sc(params, neigh_pix, neigh_weight)


# speedup vs baseline: 2.4681x; 1.0650x over previous
import jax
import jax.numpy as jnp
from jax import lax
from jax.experimental import pallas as pl
from jax.experimental.pallas import tpu as pltpu
from jax.experimental.pallas import tpu_sc as plsc

N_LEVELS = 10
F = 16
N = 100000
OUT_D = F * N_LEVELS

C = 160
CW = 4 * N_LEVELS * C
NCHUNKS = N // C
NW = 32
LANES = 16


def _heal_body(pix_hbm, w_hbm, table_hbm, out_hbm,
               pix_v, w_v, idx_v, rows_v, out_v, sem_g0, sem_g1):
    wid = lax.axis_index("s") * 2 + lax.axis_index("c")
    niter = (NCHUNKS - wid + NW - 1) // NW

    iota = lax.iota(jnp.int32, LANES)
    col_base = iota * N_LEVELS

    def compute_level(l, rbuf):
        col = col_base + l
        wbase = l * 4 * C

        def group(g, _):
            b16 = g * LANES
            wv = [w_v[pl.ds(wbase + j * C + b16, LANES)] for j in range(4)]
            for p in range(LANES):
                n = b16 + p
                acc = wv[0][p] * rbuf[n]
                acc = acc + wv[1][p] * rbuf[C + n]
                acc = acc + wv[2][p] * rbuf[2 * C + n]
                acc = acc + wv[3][p] * rbuf[3 * C + n]
                plsc.store_scatter(out_v, [col + n * OUT_D], acc)
            return 0
        lax.fori_loop(0, C // LANES, group, 0)

    def chunk_body(it, _):
        c = wid + it * NW
        base = c * C

        pltpu.sync_copy(pix_hbm.at[pl.ds(c * CW, CW)], pix_v)
        pltpu.sync_copy(w_hbm.at[pl.ds(c * CW, CW)], w_v)

        def idx_level(l, _):
            start = ((jnp.int32(1) << (2 * l)) - 1) * 4
            pbase = l * 4 * C

            def ib(i, _):
                for j in range(4):
                    v = pix_v[pl.ds(pbase + j * C + i * LANES, LANES)]
                    idx_v[l, pl.ds(j * C + i * LANES, LANES)] = v + start
                return 0
            lax.fori_loop(0, C // LANES, ib, 0)
            return 0
        lax.fori_loop(0, N_LEVELS, idx_level, 0)

        pltpu.async_copy(table_hbm.at[idx_v.at[0]], rows_v.at[0], sem_g0)

        def pair(i, _):
            l0 = 2 * i
            pltpu.async_copy(table_hbm.at[idx_v.at[l0 + 1]], rows_v.at[1],
                             sem_g1)
            pltpu.make_async_copy(table_hbm.at[idx_v.at[l0]], rows_v.at[0],
                                  sem_g0).wait()
            compute_level(l0, rows_v.at[0])

            @pl.when(i < (N_LEVELS // 2 - 1))
            def _():
                pltpu.async_copy(table_hbm.at[idx_v.at[l0 + 2]],
                                 rows_v.at[0], sem_g0)

            pltpu.make_async_copy(table_hbm.at[idx_v.at[l0 + 1]],
                                  rows_v.at[1], sem_g1).wait()
            compute_level(l0 + 1, rows_v.at[1])
            return 0
        lax.fori_loop(0, N_LEVELS // 2, pair, 0)

        pltpu.sync_copy(out_v, out_hbm.at[pl.ds(base * OUT_D, C * OUT_D)])
        return 0

    lax.fori_loop(0, niter, chunk_body, 0)


def _heal_sc(params, neigh_pix, neigh_weight):
    mesh = plsc.VectorSubcoreMesh(core_axis_name="c", subcore_axis_name="s")
    kfn = pl.kernel(
        _heal_body,
        mesh=mesh,
        out_type=jax.ShapeDtypeStruct((N * OUT_D,), jnp.float32),
        scratch_types=[
            pltpu.VMEM((CW,), jnp.int32),
            pltpu.VMEM((CW,), jnp.float32),
            pltpu.VMEM((N_LEVELS, 4 * C), jnp.int32),
            pltpu.VMEM((2, 4 * C, F), jnp.float32),
            pltpu.VMEM((C * OUT_D,), jnp.float32),
            pltpu.SemaphoreType.DMA,
            pltpu.SemaphoreType.DMA,
        ],
        compiler_params=pltpu.CompilerParams(
            use_tc_tiling_on_sc=False, needs_layout_passes=False),
    )
    pix_b = neigh_pix.reshape(N_LEVELS, 4, NCHUNKS, C)
    pix_b = pix_b.transpose(2, 0, 1, 3).reshape(-1)
    w_b = neigh_weight.reshape(N_LEVELS, 4, NCHUNKS, C)
    w_b = w_b.transpose(2, 0, 1, 3).reshape(-1)
    out = kfn(pix_b, w_b, params)
    return out.reshape(N, OUT_D)


def kernel(x, params, neigh_pix, neigh_weight):
    del x
    return _heal_sc(params, neigh_pix, neigh_weight)
